# Initial kernel scaffold; baseline (speedup 1.0000x reference)
import jax
import jax.numpy as jnp
from jax.experimental import pallas as pl


def kernel(pred, bboxes, labels, anchors):
    raise NotImplementedError("write your pallas kernel here")


# speedup vs baseline: 11.8707x; 11.8707x over previous
import jax
import jax.numpy as jnp
from jax import lax
from jax.experimental import pallas as pl

_NUM_CLASSES = 80
_L_COORD = 0.05
_L_CONF = 1.0
_L_CLS = 0.5
_B, _H, _W, _A = 16, 64, 64, 3
_N = 64
_HW = _H * _W
_CH = _A * (5 + _NUM_CLASSES)
_ROWS = _B * _HW * _A


def _softplus(x):
    return jnp.maximum(x, 0.0) + jnp.log1p(jnp.exp(-jnp.abs(x)))


def _loss_body(pred_ref, bb_ref, lab_ref, anc_ref, out_ref):
    i = pl.program_id(0)

    @pl.when(i == 0)
    def _init():
        out_ref[...] = jnp.zeros_like(out_ref)

    p = pred_ref[0]
    bb = bb_ref[0]
    lab = lab_ref[0]
    anc = anc_ref[...]

    eps = 1e-8
    cell = 1.0 / jnp.float32(_W)
    cxy = bb[:, 0:2] + bb[:, 2:4] * 0.5
    wh = bb[:, 2:4]
    cij_f = jnp.floor(cxy / cell)
    frac = (cxy - cij_f * cell) / cell + eps
    txy = -jnp.log(1.0 / frac - 1.0)

    log_wh = jnp.log(wh)
    log_anc = jnp.log(anc)
    twh0 = log_wh - log_anc[0:1, :]
    twh1 = log_wh - log_anc[1:2, :]
    twh2 = log_wh - log_anc[2:3, :]
    s0 = jnp.sum(jnp.abs(twh0), axis=1, keepdims=True)
    s1 = jnp.sum(jnp.abs(twh1), axis=1, keepdims=True)
    s2 = jnp.sum(jnp.abs(twh2), axis=1, keepdims=True)
    is0 = jnp.logical_and(s0 <= s1, s0 <= s2)
    is1 = jnp.logical_and(jnp.logical_not(is0), s1 <= s2)
    is2 = jnp.logical_not(jnp.logical_or(is0, is1))
    m0 = is0.astype(jnp.float32)
    m1 = is1.astype(jnp.float32)
    m2 = is2.astype(jnp.float32)
    sS0 = jnp.max(m0)
    sS1 = jnp.max(m1)
    sS2 = jnp.max(m2)
    n_obj_i = jnp.float32(_N) * (sS0 + sS1 + sS2)

    cellflat = (cij_f[:, 1:2] * jnp.float32(_W) + cij_f[:, 0:1]).astype(jnp.int32)
    pos = lax.broadcasted_iota(jnp.int32, (_N, _HW), 1)
    onehot = (cellflat == pos).astype(jnp.float32)
    g = jnp.dot(onehot, p, preferred_element_type=jnp.float32)

    coord = jnp.float32(0.0)
    objx = jnp.float32(0.0)
    for a, (twh_a, ss_a) in enumerate(((twh0, sS0), (twh1, sS1), (twh2, sS2))):
        base = a * (5 + _NUM_CLASSES)
        d_xy = g[:, base:base + 2] - txy
        d_wh = g[:, base + 2:base + 4] - twh_a
        coord += ss_a * (jnp.sum(d_xy * d_xy) + jnp.sum(d_wh * d_wh))
        objx += ss_a * jnp.sum(g[:, base + 4:base + 5])

    sp = (jnp.sum(_softplus(p[:, 4:5]))
          + jnp.sum(_softplus(p[:, 89:90]))
          + jnp.sum(_softplus(p[:, 174:175])))

    z = (m0 * g[:, 5:85] + m1 * g[:, 90:170] + m2 * g[:, 175:255])
    zmax = jnp.max(z, axis=1, keepdims=True)
    lse = zmax + jnp.log(jnp.sum(jnp.exp(z - zmax), axis=1, keepdims=True))
    cls_iota = lax.broadcasted_iota(jnp.int32, (_N, _NUM_CLASSES), 1)
    onehot_lab = (lab == cls_iota).astype(jnp.float32)
    z_lab = jnp.sum(onehot_lab * z, axis=1, keepdims=True)
    ce = jnp.sum(lse - z_lab)

    row = lax.broadcasted_iota(jnp.int32, (8, 128), 0)
    lane = lax.broadcasted_iota(jnp.int32, (8, 128), 1)
    contrib = jnp.where(jnp.logical_and(row == 0, lane == 0), coord, 0.0)
    contrib += jnp.where(jnp.logical_and(row == 0, lane == 1), sp, 0.0)
    contrib += jnp.where(jnp.logical_and(row == 0, lane == 2), objx, 0.0)
    contrib += jnp.where(jnp.logical_and(row == 0, lane == 3), ce, 0.0)
    contrib += jnp.where(jnp.logical_and(row == 0, lane == 4), n_obj_i, 0.0)
    out_ref[...] += contrib


def kernel(pred, bboxes, labels, anchors):
    pred_r = pred.reshape(_B, _HW, _CH)
    lab_r = labels.reshape(_B, _N, 1).astype(jnp.int32)

    out = pl.pallas_call(
        _loss_body,
        grid=(_B,),
        in_specs=[
            pl.BlockSpec((1, _HW, _CH), lambda i: (i, 0, 0)),
            pl.BlockSpec((1, _N, 4), lambda i: (i, 0, 0)),
            pl.BlockSpec((1, _N, 1), lambda i: (i, 0, 0)),
            pl.BlockSpec((_A, 2), lambda i: (0, 0)),
        ],
        out_specs=pl.BlockSpec((8, 128), lambda i: (0, 0)),
        out_shape=jax.ShapeDtypeStruct((8, 128), jnp.float32),
    )(pred_r, bboxes, lab_r, anchors)

    o = out[0]
    coord_sum, sp_sum, objx, ce_sum, n_obj = o[0], o[1], o[2], o[3], o[4]
    coord_loss = _L_COORD * coord_sum / (n_obj * 4.0)
    conf_loss = _L_CONF * (sp_sum - objx) / jnp.float32(_ROWS)
    class_loss = _L_CLS * ce_sum / n_obj
    loss = coord_loss + conf_loss + class_loss
    return (loss, coord_loss, conf_loss, class_loss)


# speedup vs baseline: 14.5880x; 1.2289x over previous
import jax
import jax.numpy as jnp
from jax import lax
from jax.experimental import pallas as pl

_NUM_CLASSES = 80
_L_COORD = 0.05
_L_CONF = 1.0
_L_CLS = 0.5
_B, _H, _W, _A = 16, 64, 64, 3
_N = 64
_HW = _H * _W
_CH = _A * (5 + _NUM_CLASSES)
_ROWS = _B * _HW * _A


def _softplus(x):
    return jnp.maximum(x, 0.0) + jnp.log1p(jnp.exp(-jnp.abs(x)))


def _loss_body(pred_ref, bb_ref, lab_ref, anc_ref, out_ref):
    i = pl.program_id(0)

    @pl.when(i == 0)
    def _init():
        out_ref[...] = jnp.zeros_like(out_ref)

    p = pred_ref[0]
    bb = bb_ref[0]
    lab = lab_ref[0]
    anc = anc_ref[...]

    eps = 1e-8
    cell = 1.0 / jnp.float32(_W)
    cxy = bb[:, 0:2] + bb[:, 2:4] * 0.5
    wh = bb[:, 2:4]
    cij_f = jnp.floor(cxy / cell)
    frac = (cxy - cij_f * cell) / cell + eps
    txy = -jnp.log(1.0 / frac - 1.0)

    log_wh = jnp.log(wh)
    log_anc = jnp.log(anc)
    twh0 = log_wh - log_anc[0:1, :]
    twh1 = log_wh - log_anc[1:2, :]
    twh2 = log_wh - log_anc[2:3, :]
    s0 = jnp.sum(jnp.abs(twh0), axis=1, keepdims=True)
    s1 = jnp.sum(jnp.abs(twh1), axis=1, keepdims=True)
    s2 = jnp.sum(jnp.abs(twh2), axis=1, keepdims=True)
    is0 = jnp.logical_and(s0 <= s1, s0 <= s2)
    is1 = jnp.logical_and(jnp.logical_not(is0), s1 <= s2)
    is2 = jnp.logical_not(jnp.logical_or(is0, is1))
    m0 = is0.astype(jnp.float32)
    m1 = is1.astype(jnp.float32)
    m2 = is2.astype(jnp.float32)
    sS0 = jnp.max(m0)
    sS1 = jnp.max(m1)
    sS2 = jnp.max(m2)
    n_obj_i = jnp.float32(_N) * (sS0 + sS1 + sS2)

    cellflat = (cij_f[:, 1:2] * jnp.float32(_W) + cij_f[:, 0:1]).astype(jnp.int32)
    pos = lax.broadcasted_iota(jnp.int32, (_N, _HW), 1)
    onehot = (cellflat == pos).astype(jnp.float32)
    g = jnp.dot(onehot, p, preferred_element_type=jnp.float32)

    coord = jnp.float32(0.0)
    objx = jnp.float32(0.0)
    for a, (twh_a, ss_a) in enumerate(((twh0, sS0), (twh1, sS1), (twh2, sS2))):
        base = a * (5 + _NUM_CLASSES)
        d_xy = g[:, base:base + 2] - txy
        d_wh = g[:, base + 2:base + 4] - twh_a
        coord += ss_a * (jnp.sum(d_xy * d_xy) + jnp.sum(d_wh * d_wh))
        objx += ss_a * jnp.sum(g[:, base + 4:base + 5])

    conf_cols = jnp.concatenate(
        [p[:, 4:5].reshape(32, 128),
         p[:, 89:90].reshape(32, 128),
         p[:, 174:175].reshape(32, 128)], axis=0)
    sp = jnp.sum(_softplus(conf_cols))

    z = (m0 * g[:, 5:85] + m1 * g[:, 90:170] + m2 * g[:, 175:255])
    zmax = jnp.max(z, axis=1, keepdims=True)
    lse = zmax + jnp.log(jnp.sum(jnp.exp(z - zmax), axis=1, keepdims=True))
    cls_iota = lax.broadcasted_iota(jnp.int32, (_N, _NUM_CLASSES), 1)
    onehot_lab = (lab == cls_iota).astype(jnp.float32)
    z_lab = jnp.sum(onehot_lab * z, axis=1, keepdims=True)
    ce = jnp.sum(lse - z_lab)

    row = lax.broadcasted_iota(jnp.int32, (8, 128), 0)
    lane = lax.broadcasted_iota(jnp.int32, (8, 128), 1)
    contrib = jnp.where(jnp.logical_and(row == 0, lane == 0), coord, 0.0)
    contrib += jnp.where(jnp.logical_and(row == 0, lane == 1), sp, 0.0)
    contrib += jnp.where(jnp.logical_and(row == 0, lane == 2), objx, 0.0)
    contrib += jnp.where(jnp.logical_and(row == 0, lane == 3), ce, 0.0)
    contrib += jnp.where(jnp.logical_and(row == 0, lane == 4), n_obj_i, 0.0)
    out_ref[...] += contrib


def kernel(pred, bboxes, labels, anchors):
    pred_r = pred.reshape(_B, _HW, _CH)
    lab_r = labels.reshape(_B, _N, 1).astype(jnp.int32)

    out = pl.pallas_call(
        _loss_body,
        grid=(_B,),
        in_specs=[
            pl.BlockSpec((1, _HW, _CH), lambda i: (i, 0, 0)),
            pl.BlockSpec((1, _N, 4), lambda i: (i, 0, 0)),
            pl.BlockSpec((1, _N, 1), lambda i: (i, 0, 0)),
            pl.BlockSpec((_A, 2), lambda i: (0, 0)),
        ],
        out_specs=pl.BlockSpec((8, 128), lambda i: (0, 0)),
        out_shape=jax.ShapeDtypeStruct((8, 128), jnp.float32),
    )(pred_r, bboxes, lab_r, anchors)

    o = out[0]
    coord_sum, sp_sum, objx, ce_sum, n_obj = o[0], o[1], o[2], o[3], o[4]
    coord_loss = _L_COORD * coord_sum / (n_obj * 4.0)
    conf_loss = _L_CONF * (sp_sum - objx) / jnp.float32(_ROWS)
    class_loss = _L_CLS * ce_sum / n_obj
    loss = coord_loss + conf_loss + class_loss
    return (loss, coord_loss, conf_loss, class_loss)
